# baseline (speedup 1.0000x reference)
import jax
import jax.numpy as jnp
from jax import lax
from jax.experimental import pallas as pl
from jax.experimental.pallas import tpu as pltpu
from jax.experimental.pallas import tpu_sc as plsc

_NUM_EMB = 1000000
_DIM = 64
_NVALS = 81920

_NC = 2
_NS = 16
_L = 16
_NW = _NC * _NS
_BPW = _NVALS // _NW
_CH = 128
_NCH = _BPW // _CH


def _body(v0, v1, t0, t1, emb_out, rem_out, idx_c, pidx_c, prows, crows, sem):
    wid = lax.axis_index("s") * _NC + lax.axis_index("c")
    base = wid * _BPW
    iota = lax.iota(jnp.int32, _L)
    for f, (vals, table) in enumerate(((v0, t0), (v1, t1))):
        def step(j, carry, vals=vals, table=table, f=f):
            off = pl.multiple_of(base + j * _CH, _CH)
            pltpu.sync_copy(vals.at[pl.ds(off, _CH)], idx_c)
            for k in range(_CH // _L):
                sl = pl.ds(k * _L, _L)
                r = idx_c[sl] % _NUM_EMB
                idx_c[sl] = r
                pidx_c[sl] = lax.shift_right_logical(r, 1)
            pltpu.sync_copy(idx_c, rem_out.at[pl.ds(f * _NVALS + off, _CH)])
            pltpu.async_copy(table.at[pidx_c], prows, sem).wait()
            for g in range(_CH // _L):
                row_vec = iota + (g * _L)
                rowflat = lax.shift_left(row_vec, 6)
                rg = idx_c[pl.ds(g * _L, _L)]
                colb = lax.shift_left(rg & 1, 6)
                for c in range(_DIM):
                    val = plsc.load_gather(prows, [row_vec, colb + c])
                    flat = rowflat + c
                    plsc.store_scatter(
                        crows,
                        [lax.shift_right_logical(flat, 7), flat & 127], val)
            pltpu.sync_copy(
                crows,
                emb_out.at[f, pl.ds(pl.multiple_of(off >> 1, _CH >> 1),
                                    _CH >> 1)])
            return carry

        lax.fori_loop(0, _NCH, step, 0)


def kernel(values_0, values_1, lengths, table_0, table_1):
    del lengths
    call = pl.kernel(
        _body,
        out_type=(
            jax.ShapeDtypeStruct((2, _NVALS // 2, 2 * _DIM), jnp.float32),
            jax.ShapeDtypeStruct((2 * _NVALS,), jnp.int32),
        ),
        mesh=plsc.VectorSubcoreMesh(core_axis_name="c", subcore_axis_name="s"),
        scratch_types=[
            pltpu.VMEM((_CH,), jnp.int32),
            pltpu.VMEM((_CH,), jnp.int32),
            pltpu.VMEM((_CH, 2 * _DIM), jnp.float32),
            pltpu.VMEM((_CH >> 1, 2 * _DIM), jnp.float32),
            pltpu.SemaphoreType.DMA,
        ],
        compiler_params=pltpu.CompilerParams(use_tc_tiling_on_sc=True, needs_layout_passes=False),
    )
    emb, rem = call(
        values_0, values_1,
        table_0.reshape(_NUM_EMB // 2, 2 * _DIM),
        table_1.reshape(_NUM_EMB // 2, 2 * _DIM),
    )
    return emb.reshape(2, _NVALS, _DIM), rem.reshape(2, _NVALS)


# speedup vs baseline: 1.3346x; 1.3346x over previous
import jax
import jax.numpy as jnp
from jax import lax
from jax.experimental import pallas as pl
from jax.experimental.pallas import tpu as pltpu
from jax.experimental.pallas import tpu_sc as plsc

_NUM_EMB = 1000000
_DIM = 64
_NVALS = 81920

_NC = 2
_NS = 16
_L = 16
_NW = _NC * _NS
_BPW = _NVALS // _NW
_CH = 128
_NCH = _BPW // _CH
_NSLOT = 8
_LEAD = 4


def _body(vals, table, emb_out, rem_out, idx2, rows, gsems, ssems):
    wid = lax.axis_index("s") * _NC + lax.axis_index("c")
    base = wid * _BPW
    pltpu.sync_copy(vals.at[wid], idx2)

    def mod_j(j, carry):
        for k in range(_CH // _L):
            sl = (j, pl.ds(k * _L, _L))
            idx2[sl] = idx2[sl] % _NUM_EMB
        return carry

    lax.fori_loop(0, _NCH, mod_j, 0)
    pltpu.sync_copy(idx2, rem_out.at[wid])

    gd = [None] * _NSLOT
    sd = [None] * _NSLOT

    def start_gather(j):
        b = j % _NSLOT
        gd[b] = pltpu.async_copy(table.at[idx2.at[j]], rows.at[b],
                                 gsems.at[b])

    for j in range(_LEAD):
        start_gather(j)
    for j in range(_NCH):
        b = j % _NSLOT
        gd[b].wait()
        sd[b] = pltpu.async_copy(
            rows.at[b], emb_out.at[pl.ds(base + j * _CH, _CH)],
            ssems.at[b])
        nj = j + _LEAD
        if nj < _NCH:
            nb = nj % _NSLOT
            if sd[nb] is not None:
                sd[nb].wait()
            start_gather(nj)
    for j in range(_NCH - _NSLOT, _NCH):
        sd[j % _NSLOT].wait()


def _feature_call(values, table):
    call = pl.kernel(
        _body,
        out_type=(
            jax.ShapeDtypeStruct((_NVALS, _DIM), jnp.float32),
            jax.ShapeDtypeStruct((_NW, _NCH, _CH), jnp.int32),
        ),
        mesh=plsc.VectorSubcoreMesh(core_axis_name="c", subcore_axis_name="s"),
        scratch_types=[
            pltpu.VMEM((_NCH, _CH), jnp.int32),
            pltpu.VMEM((_NSLOT, _CH, _DIM), jnp.float32),
            pltpu.SemaphoreType.DMA((_NSLOT,)),
            pltpu.SemaphoreType.DMA((_NSLOT,)),
        ],
        compiler_params=pltpu.CompilerParams(use_tc_tiling_on_sc=False),
    )
    return call(values.reshape(_NW, _NCH, _CH), table)


def kernel(values_0, values_1, lengths, table_0, table_1):
    del lengths
    emb0, rem0 = _feature_call(values_0, table_0)
    emb1, rem1 = _feature_call(values_1, table_1)
    emb = jnp.stack([emb0, emb1])
    rem = jnp.stack([rem0.reshape(_NVALS), rem1.reshape(_NVALS)])
    return emb, rem


# speedup vs baseline: 1.3595x; 1.0186x over previous
import jax
import jax.numpy as jnp
from jax import lax
from jax.experimental import pallas as pl
from jax.experimental.pallas import tpu as pltpu
from jax.experimental.pallas import tpu_sc as plsc

_NUM_EMB = 1000000
_DIM = 64
_NVALS = 81920

_NC = 2
_NS = 16
_L = 16
_NW = _NC * _NS
_BPW = _NVALS // _NW
_CH = 128
_NCH = _BPW // _CH
_NSLOT = 8
_LEAD = 4


def _body(v0, v1, t0, t1, emb_out, rem_out, idx1, idx2, rows, gsems, ssems):
    wid = lax.axis_index("s") * _NC + lax.axis_index("c")
    base = wid * _BPW
    for f, (vals, table) in enumerate(((v0, t0), (v1, t1))):
        pltpu.sync_copy(vals.at[pl.ds(base, _BPW)], idx1)

        def mod_j(j, carry):
            for k in range(_CH // _L):
                sl = pl.ds(j * _CH + k * _L, _L)
                r = idx1[sl] % _NUM_EMB
                idx1[sl] = r
                idx2[j, pl.ds(k * _L, _L)] = r
            return carry

        lax.fori_loop(0, _NCH, mod_j, 0)
        pltpu.sync_copy(idx1, rem_out.at[pl.ds(f * _NVALS + base, _BPW)])

        gd = [None] * _NSLOT
        sd = [None] * _NSLOT

        def start_gather(j, table=table):
            b = j % _NSLOT
            gd[b] = pltpu.async_copy(table.at[idx2.at[j]], rows.at[b],
                                     gsems.at[b])

        for j in range(_LEAD):
            start_gather(j)
        for j in range(_NCH):
            b = j % _NSLOT
            gd[b].wait()
            sd[b] = pltpu.async_copy(
                rows.at[b], emb_out.at[f, pl.ds(base + j * _CH, _CH)],
                ssems.at[b])
            nj = j + _LEAD
            if nj < _NCH:
                nb = nj % _NSLOT
                if sd[nb] is not None:
                    sd[nb].wait()
                start_gather(nj)
        for j in range(_NCH - _NSLOT, _NCH):
            sd[j % _NSLOT].wait()


def kernel(values_0, values_1, lengths, table_0, table_1):
    del lengths
    call = pl.kernel(
        _body,
        out_type=(
            jax.ShapeDtypeStruct((2, _NVALS, _DIM), jnp.float32),
            jax.ShapeDtypeStruct((2 * _NVALS,), jnp.int32),
        ),
        mesh=plsc.VectorSubcoreMesh(core_axis_name="c", subcore_axis_name="s"),
        scratch_types=[
            pltpu.VMEM((_BPW,), jnp.int32),
            pltpu.VMEM((_NCH, _CH), jnp.int32),
            pltpu.VMEM((_NSLOT, _CH, _DIM), jnp.float32),
            pltpu.SemaphoreType.DMA((_NSLOT,)),
            pltpu.SemaphoreType.DMA((_NSLOT,)),
        ],
        compiler_params=pltpu.CompilerParams(use_tc_tiling_on_sc=False),
    )
    emb, rem = call(values_0, values_1, table_0, table_1)
    return emb, rem.reshape(2, _NVALS)


# speedup vs baseline: 1.4790x; 1.0879x over previous
import jax
import jax.numpy as jnp
from jax import lax
from jax.experimental import pallas as pl
from jax.experimental.pallas import tpu as pltpu
from jax.experimental.pallas import tpu_sc as plsc

_NUM_EMB = 1000000
_DIM = 64
_PAD = 128
_NVALS = 81920

_NC = 2
_NS = 16
_L = 16
_NW = _NC * _NS
_BPW = _NVALS // _NW
_CH = 128
_NCH = _BPW // _CH
_NSLOT = 6
_LEAD = 3


def _body(v0, v1, t0, t1, emb_out, rem_out, idx1, idx2, rows, gsems, ssems):
    wid = lax.axis_index("s") * _NC + lax.axis_index("c")
    base = pl.multiple_of(wid * _BPW, _BPW)
    for f, (vals, table) in enumerate(((v0, t0), (v1, t1))):
        pltpu.sync_copy(vals.at[pl.ds(base, _BPW)], idx1)

        def mod_j(j, carry):
            for k in range(_CH // _L):
                sl = pl.ds(j * _CH + k * _L, _L)
                r = idx1[sl] % _NUM_EMB
                idx1[sl] = r
                idx2[j, pl.ds(k * _L, _L)] = r
            return carry

        lax.fori_loop(0, _NCH, mod_j, 0)
        pltpu.sync_copy(idx1, rem_out.at[pl.ds(f * _NVALS + base, _BPW)])

        gd = [None] * _NSLOT
        sd = [None] * _NSLOT

        def start_gather(j, table=table):
            b = j % _NSLOT
            gd[b] = pltpu.async_copy(table.at[idx2.at[j]], rows.at[b],
                                     gsems.at[b])

        for j in range(_LEAD):
            start_gather(j)
        for j in range(_NCH):
            b = j % _NSLOT
            gd[b].wait()
            off = pl.multiple_of(base + j * _CH, _CH)
            sd[b] = pltpu.async_copy(
                rows.at[b], emb_out.at[f, pl.ds(off, _CH)], ssems.at[b])
            nj = j + _LEAD
            if nj < _NCH:
                nb = nj % _NSLOT
                if sd[nb] is not None:
                    sd[nb].wait()
                start_gather(nj)
        for j in range(_NCH - _NSLOT, _NCH):
            if sd[j % _NSLOT] is not None:
                sd[j % _NSLOT].wait()


def kernel(values_0, values_1, lengths, table_0, table_1):
    del lengths
    call = pl.kernel(
        _body,
        out_type=(
            jax.ShapeDtypeStruct((2, _NVALS, _PAD), jnp.float32),
            jax.ShapeDtypeStruct((2 * _NVALS,), jnp.int32),
        ),
        mesh=plsc.VectorSubcoreMesh(core_axis_name="c", subcore_axis_name="s"),
        scratch_types=[
            pltpu.VMEM((_BPW,), jnp.int32),
            pltpu.VMEM((_NCH, _CH), jnp.int32),
            pltpu.VMEM((_NSLOT, _CH, _PAD), jnp.float32),
            pltpu.SemaphoreType.DMA((_NSLOT,)),
            pltpu.SemaphoreType.DMA((_NSLOT,)),
        ],
        compiler_params=pltpu.CompilerParams(
            use_tc_tiling_on_sc=True, needs_layout_passes=False),
    )
    tp0 = jnp.pad(table_0, ((0, 0), (0, _PAD - _DIM)))
    tp1 = jnp.pad(table_1, ((0, 0), (0, _PAD - _DIM)))
    emb, rem = call(values_0, values_1, tp0, tp1)
    return emb[:, :, :_DIM], rem.reshape(2, _NVALS)


# speedup vs baseline: 1.4820x; 1.0020x over previous
import jax
import jax.numpy as jnp
from jax import lax
from jax.experimental import pallas as pl
from jax.experimental.pallas import tpu as pltpu
from jax.experimental.pallas import tpu_sc as plsc

_NUM_EMB = 1000000
_DIM = 64
_PAD = 128
_NVALS = 81920

_NC = 2
_NS = 16
_L = 16
_NW = _NC * _NS
_BPW = _NVALS // _NW
_CH = 128
_NCH = _BPW // _CH
_NSLOT = 6
_LEAD = 3


def _body(v0, v1, t0, t1, emb_out, rem_out, idx1, idx2, rows, gsems, ssems):
    wid = lax.axis_index("s") * _NC + lax.axis_index("c")
    base = pl.multiple_of(wid * _BPW, _BPW)
    for f, (vals, table) in enumerate(((v0, t0), (v1, t1))):
        pltpu.sync_copy(vals.at[pl.ds(base, _BPW)], idx1)

        def mod_j(j, carry):
            for k in range(_CH // _L):
                sl = pl.ds(j * _CH + k * _L, _L)
                r = idx1[sl] % _NUM_EMB
                idx1[sl] = r
                idx2[j, pl.ds(k * _L, _L)] = r
            return carry

        lax.fori_loop(0, _NCH, mod_j, 0)
        pltpu.sync_copy(idx1, rem_out.at[pl.ds(f * _NVALS + base, _BPW)])

        gd = [None] * _NSLOT
        sd = [None] * _NSLOT

        def start_gather(j, table=table):
            b = j % _NSLOT
            gd[b] = pltpu.async_copy(table.at[idx2.at[j]], rows.at[b],
                                     gsems.at[b])

        for j in range(_LEAD):
            start_gather(j)
        for j in range(_NCH):
            b = j % _NSLOT
            gd[b].wait()
            off = pl.multiple_of(base + j * _CH, _CH)
            sd[b] = pltpu.async_copy(
                rows.at[b], emb_out.at[f, pl.ds(off, _CH)], ssems.at[b])
            nj = j + _LEAD
            if nj < _NCH:
                nb = nj % _NSLOT
                if sd[nb] is not None:
                    sd[nb].wait()
                start_gather(nj)
        for j in range(_NCH - _NSLOT, _NCH):
            if sd[j % _NSLOT] is not None:
                sd[j % _NSLOT].wait()


def kernel(values_0, values_1, lengths, table_0, table_1):
    del lengths
    call = pl.kernel(
        _body,
        out_type=(
            jax.ShapeDtypeStruct((2, _NVALS, _PAD), jnp.float32),
            jax.ShapeDtypeStruct((2 * _NVALS,), jnp.int32),
        ),
        mesh=plsc.VectorSubcoreMesh(core_axis_name="c", subcore_axis_name="s"),
        scratch_types=[
            pltpu.VMEM((_BPW,), jnp.int32),
            pltpu.VMEM((_NCH, _CH), jnp.int32),
            pltpu.VMEM((_NSLOT, _CH, _PAD), jnp.float32),
            pltpu.SemaphoreType.DMA((_NSLOT,)),
            pltpu.SemaphoreType.DMA((_NSLOT,)),
        ],
        compiler_params=pltpu.CompilerParams(
            use_tc_tiling_on_sc=True, needs_layout_passes=False),
    )
    z = jnp.zeros((_NUM_EMB, _PAD - _DIM), jnp.float32)
    tp0 = jnp.concatenate([table_0, z], axis=1)
    tp1 = jnp.concatenate([table_1, z], axis=1)
    emb, rem = call(values_0, values_1, tp0, tp1)
    return emb[:, :, :_DIM], rem.reshape(2, _NVALS)
